# Initial kernel scaffold; baseline (speedup 1.0000x reference)
import jax
import jax.numpy as jnp
from jax.experimental import pallas as pl


def kernel(own_move_idx, own_item_idx, own_ability_idx, own_type_idx, own_status_idx, opp_move_idx, opp_item_idx, opp_ability_idx, opp_type_idx, opp_status_idx, field_attrib_idx, side_attrib_idx, opp_side_attrib_idx, own_stats, own_boosts, opp_stats, opp_boosts, dyn_flags, W_move, W_item, W_ability, W_side, W_field, W_type, W_status):
    raise NotImplementedError("write your pallas kernel here")


# speedup vs baseline: 6.2294x; 6.2294x over previous
import jax
import jax.numpy as jnp
from jax import lax
from jax.experimental import pallas as pl
from jax.experimental.pallas import tpu as pltpu
from jax.experimental.pallas import tpu_sc as plsc

B = 16384
NC, NS, L = 2, 16, 16
NW = NC * NS
BW = B // NW
BC = 128
G = BC // L
NSUB = BW // BC

MOVE_DIM, ITEM_DIM, ABILITY_DIM = 32, 32, 16
SIDE_DIM, FIELD_DIM, TYPE_DIM, STATUS_DIM = 16, 16, 16, 8

OWN_BASE = 34
OWN_PER = 214
MID_BASE = OWN_BASE + 6 * OWN_PER
OPP_BASE = MID_BASE + 18
OPP_PER = 118
D_OUT = OPP_BASE + 6 * OPP_PER


def _sc_encode(mv_own, it_own, ab_own, ty_own, st_own,
               mv_opp, it_opp, ab_opp, ty_opp, st_opp,
               fld, sd, osd, stats_own, boosts_own, stats_opp, boosts_opp,
               dyn, W_move, W_item, W_ability, W_side, W_field, W_type,
               W_status, out,
               tab_ab, tab_sd, tab_fl, tab_ty, tab_st,
               rows_buf, idx_buf, tile, sem):
    cid = lax.axis_index("c")
    sid = lax.axis_index("s")
    wid = sid * NC + cid

    iota = lax.iota(jnp.int32, L)
    row_ids = [iota + g * L for g in range(G)]

    pltpu.sync_copy(W_ability, tab_ab)
    pltpu.sync_copy(W_side, tab_sd)
    pltpu.sync_copy(W_field, tab_fl)
    pltpu.sync_copy(W_type, tab_ty)
    pltpu.sync_copy(W_status, tab_st)

    def transpose_rows(srcs, n_d, tile_base, scale):
        def dbody(d, _):
            col = jnp.full((L,), d, jnp.int32)
            for g in range(G):
                v = plsc.load_gather(srcs[0], [row_ids[g], col])
                for s in srcs[1:]:
                    v = v + plsc.load_gather(s, [row_ids[g], col])
                if scale != 1.0:
                    v = v * scale
                tile[tile_base + d, pl.ds(g * L, L)] = v
            return 0
        lax.fori_loop(0, n_d, dbody, 0)

    def small_lookup(tab, idx_rows, n_d, tile_base, scale):
        def dbody(d, _):
            col = jnp.full((L,), d, jnp.int32)
            for g in range(G):
                regs = [r[g] for r in idx_rows]
                v = plsc.load_gather(tab, [regs[0], col])
                for r in regs[1:]:
                    v = v + plsc.load_gather(tab, [r, col])
                if scale != 1.0:
                    v = v * scale
                tile[tile_base + d, pl.ds(g * L, L)] = v
            return 0
        lax.fori_loop(0, n_d, dbody, 0)

    def load_idx_regs(n_rows):
        return [[idx_buf[j, pl.ds(g * L, L)] for g in range(G)]
                for j in range(n_rows)]

    def scale_tile_rows(tile_base, n_d, scale):
        def dbody(d, _):
            for g in range(G):
                sl = pl.ds(g * L, L)
                tile[tile_base + d, sl] = tile[tile_base + d, sl] * scale
            return 0
        lax.fori_loop(0, n_d, dbody, 0)

    def sub_body(sub, _):
        b0 = wid * BW + sub * BC
        bsl = pl.ds(b0, BC)

        pltpu.sync_copy(fld.at[:, bsl], idx_buf.at[pl.ds(0, 3)])
        small_lookup(tab_fl, load_idx_regs(3), FIELD_DIM, 0, 1.0)
        pltpu.sync_copy(dyn.at[pl.ds(0, 2), bsl], tile.at[pl.ds(16, 2), :])
        pltpu.sync_copy(sd.at[:, bsl], idx_buf.at[pl.ds(0, 4)])
        small_lookup(tab_sd, load_idx_regs(4), SIDE_DIM, 18, 1.0)
        pltpu.sync_copy(tile.at[pl.ds(0, 34), :],
                        out.at[pl.ds(0, 34), bsl])

        pltpu.sync_copy(dyn.at[pl.ds(2, 2), bsl], tile.at[pl.ds(0, 2), :])
        pltpu.sync_copy(osd.at[:, bsl], idx_buf.at[pl.ds(0, 4)])
        small_lookup(tab_sd, load_idx_regs(4), SIDE_DIM, 2, 1.0)
        pltpu.sync_copy(tile.at[pl.ds(0, 18), :],
                        out.at[pl.ds(MID_BASE, 18), bsl])

        def own_body(p, _):
            pltpu.sync_copy(mv_own.at[p, :, bsl], idx_buf)
            for m in range(4):
                pltpu.async_copy(W_move.at[idx_buf.at[m]],
                                 rows_buf.at[m], sem)
            for m in range(4):
                pltpu.make_async_copy(W_move.at[idx_buf.at[m]],
                                      rows_buf.at[m], sem).wait()
            for m in range(4):
                transpose_rows([rows_buf.at[m]], MOVE_DIM, m * 32, 1.0)

            pltpu.sync_copy(it_own.at[p, :, bsl], idx_buf.at[pl.ds(0, 2)])
            for m in range(2):
                pltpu.async_copy(W_item.at[idx_buf.at[m]],
                                 rows_buf.at[m], sem)
            for m in range(2):
                pltpu.make_async_copy(W_item.at[idx_buf.at[m]],
                                      rows_buf.at[m], sem).wait()
            transpose_rows([rows_buf.at[0], rows_buf.at[1]],
                           ITEM_DIM, 128, 0.5)

            pltpu.sync_copy(ab_own.at[p, :, bsl], idx_buf.at[pl.ds(0, 2)])
            small_lookup(tab_ab, load_idx_regs(2), ABILITY_DIM, 160, 0.5)
            pltpu.sync_copy(ty_own.at[p, :, bsl], idx_buf.at[pl.ds(0, 2)])
            small_lookup(tab_ty, load_idx_regs(2), TYPE_DIM, 176, 1.0)
            pltpu.sync_copy(st_own.at[p, bsl], idx_buf.at[0])
            small_lookup(tab_st, load_idx_regs(1), STATUS_DIM, 192, 1.0)

            pltpu.sync_copy(stats_own.at[p, :, bsl],
                            tile.at[pl.ds(200, 7), :])
            pltpu.sync_copy(boosts_own.at[p, :, bsl],
                            tile.at[pl.ds(207, 7), :])
            scale_tile_rows(207, 7, 1.0 / 6.0)

            pltpu.sync_copy(tile.at[pl.ds(0, OWN_PER), :],
                            out.at[pl.ds(OWN_BASE + p * OWN_PER, OWN_PER),
                                   bsl])
            return 0
        lax.fori_loop(0, 6, own_body, 0)

        def opp_body(p, _):
            pltpu.sync_copy(mv_opp.at[p, :, bsl], idx_buf)
            for m in range(4):
                pltpu.async_copy(W_move.at[idx_buf.at[m]],
                                 rows_buf.at[m], sem)
            for m in range(4):
                pltpu.make_async_copy(W_move.at[idx_buf.at[m]],
                                      rows_buf.at[m], sem).wait()
            transpose_rows([rows_buf.at[m] for m in range(4)],
                           MOVE_DIM, 0, 0.25)

            pltpu.sync_copy(it_opp.at[p, :, bsl], idx_buf.at[pl.ds(0, 2)])
            for m in range(2):
                pltpu.async_copy(W_item.at[idx_buf.at[m]],
                                 rows_buf.at[m], sem)
            for m in range(2):
                pltpu.make_async_copy(W_item.at[idx_buf.at[m]],
                                      rows_buf.at[m], sem).wait()
            transpose_rows([rows_buf.at[0], rows_buf.at[1]],
                           ITEM_DIM, 32, 0.5)

            pltpu.sync_copy(ab_opp.at[p, :, bsl], idx_buf.at[pl.ds(0, 2)])
            small_lookup(tab_ab, load_idx_regs(2), ABILITY_DIM, 64, 0.5)
            pltpu.sync_copy(ty_opp.at[p, :, bsl], idx_buf.at[pl.ds(0, 2)])
            small_lookup(tab_ty, load_idx_regs(2), TYPE_DIM, 80, 1.0)
            pltpu.sync_copy(st_opp.at[p, bsl], idx_buf.at[0])
            small_lookup(tab_st, load_idx_regs(1), STATUS_DIM, 96, 1.0)

            pltpu.sync_copy(stats_opp.at[p, :, bsl],
                            tile.at[pl.ds(104, 7), :])
            pltpu.sync_copy(boosts_opp.at[p, :, bsl],
                            tile.at[pl.ds(111, 7), :])
            scale_tile_rows(111, 7, 1.0 / 6.0)

            pltpu.sync_copy(tile.at[pl.ds(0, OPP_PER), :],
                            out.at[pl.ds(OPP_BASE + p * OPP_PER, OPP_PER),
                                   bsl])
            return 0
        lax.fori_loop(0, 6, opp_body, 0)
        return 0

    lax.fori_loop(0, NSUB, sub_body, 0)


@jax.jit
def kernel(own_move_idx, own_item_idx, own_ability_idx, own_type_idx,
           own_status_idx, opp_move_idx, opp_item_idx, opp_ability_idx,
           opp_type_idx, opp_status_idx, field_attrib_idx, side_attrib_idx,
           opp_side_attrib_idx, own_stats, own_boosts, opp_stats, opp_boosts,
           dyn_flags, W_move, W_item, W_ability, W_side, W_field, W_type,
           W_status):
    t3 = lambda a: jnp.transpose(a, (1, 2, 0)).astype(jnp.int32)
    f3 = lambda a: jnp.transpose(a, (1, 2, 0))
    mesh = plsc.VectorSubcoreMesh(core_axis_name="c", subcore_axis_name="s",
                                  num_cores=NC, num_subcores=NS)
    run = pl.kernel(
        _sc_encode,
        out_type=jax.ShapeDtypeStruct((D_OUT, B), jnp.float32),
        mesh=mesh,
        scratch_types=[
            pltpu.VMEM((1000, ABILITY_DIM), jnp.float32),
            pltpu.VMEM((64, SIDE_DIM), jnp.float32),
            pltpu.VMEM((64, FIELD_DIM), jnp.float32),
            pltpu.VMEM((32, TYPE_DIM), jnp.float32),
            pltpu.VMEM((16, STATUS_DIM), jnp.float32),
            pltpu.VMEM((4, BC, 32), jnp.float32),
            pltpu.VMEM((4, BC), jnp.int32),
            pltpu.VMEM((OWN_PER, BC), jnp.float32),
            pltpu.SemaphoreType.DMA,
        ],
        compiler_params=pltpu.CompilerParams(use_tc_tiling_on_sc=False,
                                            needs_layout_passes=False),
    )
    return run(
        t3(own_move_idx), t3(own_item_idx), t3(own_ability_idx),
        t3(own_type_idx), jnp.transpose(own_status_idx).astype(jnp.int32),
        t3(opp_move_idx), t3(opp_item_idx), t3(opp_ability_idx),
        t3(opp_type_idx), jnp.transpose(opp_status_idx).astype(jnp.int32),
        jnp.transpose(field_attrib_idx).astype(jnp.int32),
        jnp.transpose(side_attrib_idx).astype(jnp.int32),
        jnp.transpose(opp_side_attrib_idx).astype(jnp.int32),
        f3(own_stats), f3(own_boosts), f3(opp_stats), f3(opp_boosts),
        jnp.transpose(dyn_flags),
        W_move, W_item, W_ability, W_side, W_field, W_type, W_status)


# speedup vs baseline: 6.8373x; 1.0976x over previous
import jax
import jax.numpy as jnp
from jax import lax
from jax.experimental import pallas as pl
from jax.experimental.pallas import tpu as pltpu
from jax.experimental.pallas import tpu_sc as plsc

B = 16384
NC, NS, L = 2, 16, 16
NW = NC * NS
BW = B // NW
BC = 128
G = BC // L
NSUB = BW // BC

MOVE_DIM, ITEM_DIM, ABILITY_DIM = 32, 32, 16
SIDE_DIM, FIELD_DIM, TYPE_DIM, STATUS_DIM = 16, 16, 16, 8

OWN_BASE = 34
OWN_PER = 214
MID_BASE = OWN_BASE + 6 * OWN_PER
OPP_BASE = MID_BASE + 18
OPP_PER = 118
D_OUT = OPP_BASE + 6 * OPP_PER


def _sc_encode(mv_own, it_own, ab_own, ty_own, st_own,
               mv_opp, it_opp, ab_opp, ty_opp, st_opp,
               fld, sd, osd, stats_own, boosts_own, stats_opp, boosts_opp,
               dyn, W_move, W_item, W_ability, W_side, W_field, W_type,
               W_status, out,
               tab_ab, tab_sd, tab_fl, tab_ty, tab_st,
               rows, mvo_i, ito_i, abo_i, tyo_i, sto_i,
               mvp_i, itp_i, abp_i, typ_i, stp_i, hd_i, tile,
               sem_g, sem_idx, sem_out, sem_st):
    cid = lax.axis_index("c")
    sid = lax.axis_index("s")
    wid = sid * NC + cid

    iota = lax.iota(jnp.int32, L)

    pltpu.sync_copy(W_ability, tab_ab)
    pltpu.sync_copy(W_side, tab_sd)
    pltpu.sync_copy(W_field, tab_fl)
    pltpu.sync_copy(W_type, tab_ty)
    pltpu.sync_copy(W_status, tab_st)

    def gather_descs(mv_i, it_i, p):
        d = [pltpu.make_async_copy(W_move.at[mv_i.at[p, m]],
                                   rows.at[m], sem_g) for m in range(4)]
        d += [pltpu.make_async_copy(W_item.at[it_i.at[p, j]],
                                    rows.at[4 + j], sem_g) for j in range(2)]
        return d

    def lookup(tab, idx_ref, idx_rows, n_d, tile_base, scale):
        def body(t, _):
            gl = (t & (G - 1)) * L
            d = t >> 3
            col = jnp.full((L,), d, jnp.int32)
            v = None
            for j in idx_rows:
                r = idx_ref[j, pl.ds(gl, L)]
                x = plsc.load_gather(tab, [r, col])
                v = x if v is None else v + x
            if scale != 1.0:
                v = v * scale
            tile[tile_base + d, pl.ds(gl, L)] = v
            return 0
        lax.fori_loop(0, G * n_d, body, 0)

    def transpose_moves_items(sep_moves, mv_base, it_base):
        def body(t, _):
            gl = (t & (G - 1)) * L
            d = t >> 3
            rid = iota + gl
            col = jnp.full((L,), d, jnp.int32)
            sl = pl.ds(gl, L)
            if sep_moves:
                for m in range(4):
                    tile[mv_base + m * 32 + d, sl] = plsc.load_gather(
                        rows.at[m], [rid, col])
            else:
                v = plsc.load_gather(rows.at[0], [rid, col])
                for m in range(1, 4):
                    v = v + plsc.load_gather(rows.at[m], [rid, col])
                tile[mv_base + d, sl] = v * 0.25
            vi = (plsc.load_gather(rows.at[4], [rid, col]) +
                  plsc.load_gather(rows.at[5], [rid, col]))
            tile[it_base + d, sl] = vi * 0.5
            return 0
        lax.fori_loop(0, G * 32, body, 0)

    def scale_rows(tile_base, n_d, scale):
        def body(t, _):
            gl = (t & (G - 1)) * L
            d = t >> 3
            sl = pl.ds(gl, L)
            tile[tile_base + d, sl] = tile[tile_base + d, sl] * scale
            return 0
        lax.fori_loop(0, G * n_d, body, 0)

    def out_desc(n_rows, row0, bsl):
        return pltpu.make_async_copy(tile.at[pl.ds(0, n_rows), :],
                                     out.at[pl.ds(row0, n_rows), bsl],
                                     sem_out)

    def stage(p, bsl, own):
        if own:
            mv_i, it_i, ab_i, ty_i, st_i = mvo_i, ito_i, abo_i, tyo_i, sto_i
            st_src, bo_src = stats_own, boosts_own
            ab_b, ty_b, stat_b, sa_b, bo_b = 160, 176, 192, 200, 207
            n_rows = OWN_PER
            row0 = OWN_BASE + p * OWN_PER
        else:
            mv_i, it_i, ab_i, ty_i, st_i = mvp_i, itp_i, abp_i, typ_i, stp_i
            st_src, bo_src = stats_opp, boosts_opp
            ab_b, ty_b, stat_b, sa_b, bo_b = 64, 80, 96, 104, 111
            n_rows = OPP_PER
            row0 = OPP_BASE + p * OPP_PER

        g_descs = gather_descs(mv_i, it_i, p)
        for dsc in g_descs:
            dsc.start()

        @pl.when(p > 0)
        def _():
            out_desc(n_rows, row0, bsl).wait()

        stat_descs = [
            pltpu.make_async_copy(st_src.at[p, :, bsl],
                                  tile.at[pl.ds(sa_b, 7), :], sem_st),
            pltpu.make_async_copy(bo_src.at[p, :, bsl],
                                  tile.at[pl.ds(bo_b, 7), :], sem_st),
        ]
        for dsc in stat_descs:
            dsc.start()

        lookup(tab_ab, ab_i.at[p], [0, 1], ABILITY_DIM, ab_b, 0.5)
        lookup(tab_ty, ty_i.at[p], [0, 1], TYPE_DIM, ty_b, 1.0)
        lookup(tab_st, st_i, [p], STATUS_DIM, stat_b, 1.0)

        for dsc in g_descs:
            dsc.wait()
        if own:
            transpose_moves_items(True, 0, 128)
        else:
            transpose_moves_items(False, 0, 32)

        for dsc in stat_descs:
            dsc.wait()
        scale_rows(bo_b, 7, 1.0 / 6.0)

        out_desc(n_rows, row0, bsl).start()

    def sub_body(sub, _):
        b0 = wid * BW + sub * BC
        bsl = pl.ds(b0, BC)

        idx_descs = [
            pltpu.make_async_copy(mv_own.at[:, :, bsl], mvo_i, sem_idx),
            pltpu.make_async_copy(it_own.at[:, :, bsl], ito_i, sem_idx),
            pltpu.make_async_copy(ab_own.at[:, :, bsl], abo_i, sem_idx),
            pltpu.make_async_copy(ty_own.at[:, :, bsl], tyo_i, sem_idx),
            pltpu.make_async_copy(st_own.at[:, bsl], sto_i, sem_idx),
            pltpu.make_async_copy(mv_opp.at[:, :, bsl], mvp_i, sem_idx),
            pltpu.make_async_copy(it_opp.at[:, :, bsl], itp_i, sem_idx),
            pltpu.make_async_copy(ab_opp.at[:, :, bsl], abp_i, sem_idx),
            pltpu.make_async_copy(ty_opp.at[:, :, bsl], typ_i, sem_idx),
            pltpu.make_async_copy(st_opp.at[:, bsl], stp_i, sem_idx),
            pltpu.make_async_copy(fld.at[:, bsl], hd_i.at[pl.ds(0, 3)],
                                  sem_idx),
            pltpu.make_async_copy(sd.at[:, bsl], hd_i.at[pl.ds(3, 4)],
                                  sem_idx),
            pltpu.make_async_copy(osd.at[:, bsl], hd_i.at[pl.ds(7, 4)],
                                  sem_idx),
        ]
        for dsc in idx_descs:
            dsc.start()
        for dsc in idx_descs:
            dsc.wait()

        lookup(tab_fl, hd_i, [0, 1, 2], FIELD_DIM, 0, 1.0)
        lookup(tab_sd, hd_i, [3, 4, 5, 6], SIDE_DIM, 18, 1.0)
        lookup(tab_sd, hd_i, [7, 8, 9, 10], SIDE_DIM, 36, 1.0)
        pltpu.sync_copy(dyn.at[pl.ds(0, 2), bsl], tile.at[pl.ds(16, 2), :])
        pltpu.sync_copy(dyn.at[pl.ds(2, 2), bsl], tile.at[pl.ds(34, 2), :])
        hm_descs = [
            pltpu.make_async_copy(tile.at[pl.ds(0, 34), :],
                                  out.at[pl.ds(0, 34), bsl], sem_out),
            pltpu.make_async_copy(tile.at[pl.ds(34, 18), :],
                                  out.at[pl.ds(MID_BASE, 18), bsl], sem_out),
        ]
        for dsc in hm_descs:
            dsc.start()
        for dsc in hm_descs:
            dsc.wait()

        def own_body(p, _):
            stage(p, bsl, True)
            return 0
        lax.fori_loop(0, 6, own_body, 0)
        out_desc(OWN_PER, OWN_BASE + 5 * OWN_PER, bsl).wait()

        def opp_body(p, _):
            stage(p, bsl, False)
            return 0
        lax.fori_loop(0, 6, opp_body, 0)
        out_desc(OPP_PER, OPP_BASE + 5 * OPP_PER, bsl).wait()
        return 0

    lax.fori_loop(0, NSUB, sub_body, 0)


@jax.jit
def kernel(own_move_idx, own_item_idx, own_ability_idx, own_type_idx,
           own_status_idx, opp_move_idx, opp_item_idx, opp_ability_idx,
           opp_type_idx, opp_status_idx, field_attrib_idx, side_attrib_idx,
           opp_side_attrib_idx, own_stats, own_boosts, opp_stats, opp_boosts,
           dyn_flags, W_move, W_item, W_ability, W_side, W_field, W_type,
           W_status):
    t3 = lambda a: jnp.transpose(a, (1, 2, 0)).astype(jnp.int32)
    f3 = lambda a: jnp.transpose(a, (1, 2, 0))
    mesh = plsc.VectorSubcoreMesh(core_axis_name="c", subcore_axis_name="s",
                                  num_cores=NC, num_subcores=NS)
    run = pl.kernel(
        _sc_encode,
        out_type=jax.ShapeDtypeStruct((D_OUT, B), jnp.float32),
        mesh=mesh,
        scratch_types=[
            pltpu.VMEM((1000, ABILITY_DIM), jnp.float32),
            pltpu.VMEM((64, SIDE_DIM), jnp.float32),
            pltpu.VMEM((64, FIELD_DIM), jnp.float32),
            pltpu.VMEM((32, TYPE_DIM), jnp.float32),
            pltpu.VMEM((16, STATUS_DIM), jnp.float32),
            pltpu.VMEM((6, BC, 32), jnp.float32),
            pltpu.VMEM((6, 4, BC), jnp.int32),
            pltpu.VMEM((6, 2, BC), jnp.int32),
            pltpu.VMEM((6, 2, BC), jnp.int32),
            pltpu.VMEM((6, 2, BC), jnp.int32),
            pltpu.VMEM((6, BC), jnp.int32),
            pltpu.VMEM((6, 4, BC), jnp.int32),
            pltpu.VMEM((6, 2, BC), jnp.int32),
            pltpu.VMEM((6, 2, BC), jnp.int32),
            pltpu.VMEM((6, 2, BC), jnp.int32),
            pltpu.VMEM((6, BC), jnp.int32),
            pltpu.VMEM((12, BC), jnp.int32),
            pltpu.VMEM((OWN_PER, BC), jnp.float32),
            pltpu.SemaphoreType.DMA,
            pltpu.SemaphoreType.DMA,
            pltpu.SemaphoreType.DMA,
            pltpu.SemaphoreType.DMA,
        ],
        compiler_params=pltpu.CompilerParams(use_tc_tiling_on_sc=False,
                                             needs_layout_passes=False),
    )
    return run(
        t3(own_move_idx), t3(own_item_idx), t3(own_ability_idx),
        t3(own_type_idx), jnp.transpose(own_status_idx).astype(jnp.int32),
        t3(opp_move_idx), t3(opp_item_idx), t3(opp_ability_idx),
        t3(opp_type_idx), jnp.transpose(opp_status_idx).astype(jnp.int32),
        jnp.transpose(field_attrib_idx).astype(jnp.int32),
        jnp.transpose(side_attrib_idx).astype(jnp.int32),
        jnp.transpose(opp_side_attrib_idx).astype(jnp.int32),
        f3(own_stats), f3(own_boosts), f3(opp_stats), f3(opp_boosts),
        jnp.transpose(dyn_flags),
        W_move, W_item, W_ability, W_side, W_field, W_type, W_status)


# speedup vs baseline: 14.0638x; 2.0569x over previous
import jax
import jax.numpy as jnp
from jax import lax
from jax.experimental import pallas as pl
from jax.experimental.pallas import tpu as pltpu
from jax.experimental.pallas import tpu_sc as plsc

B = 16384
NC, NS, L = 2, 16, 16
NW = NC * NS
BW = B // NW
BC = 128
G = BC // L
NSUB = BW // BC

MOVE_DIM, ITEM_DIM, ABILITY_DIM = 32, 32, 16
SIDE_DIM, FIELD_DIM, TYPE_DIM, STATUS_DIM = 16, 16, 16, 8

OWN_BASE = 34
OWN_PER = 214
MID_BASE = OWN_BASE + 6 * OWN_PER
OPP_BASE = MID_BASE + 18
OPP_PER = 118
D_OUT = OPP_BASE + 6 * OPP_PER


def _sc_encode(rmv_own, rit_own, rab_own, rty_own, rst_own,
               rmv_opp, rit_opp, rab_opp, rty_opp, rst_opp,
               rfld, rsd, rosd, stats_own, boosts_own, stats_opp, boosts_opp,
               rdyn, W_move, W_item, W_ability, W_side, W_field, W_type,
               W_status, out,
               tab_ab, tab_sd, tab_fl, tab_ty, tab_st,
               rows, mv_c, it_c, ab_c, ty_c, st_c, op_mv, op_it, op_ab,
               op_ty, op_st, hd_i,
               raw24, raw12a, raw12b, raw12c, raw6,
               rawh1, rawh2, rawh3, rawh4, tile,
               sem_ga, sem_gb, sem_idx, sem_out, sem_st):
    cid = lax.axis_index("c")
    sid = lax.axis_index("s")
    wid = sid * NC + cid

    iota = lax.iota(jnp.int32, L)

    pltpu.sync_copy(W_ability, tab_ab.at[:, pl.ds(0, ABILITY_DIM)])
    pltpu.sync_copy(W_side, tab_sd.at[:, pl.ds(0, SIDE_DIM)])
    pltpu.sync_copy(W_field, tab_fl.at[:, pl.ds(0, FIELD_DIM)])
    pltpu.sync_copy(W_type, tab_ty.at[:, pl.ds(0, TYPE_DIM)])
    pltpu.sync_copy(W_status, tab_st.at[:, pl.ds(0, STATUS_DIM)])

    def deinterleave(raw_ref, n_c, dst_ref):
        @plsc.parallel_loop(0, n_c * G, unroll=4)
        def body(t):
            c = t >> 3
            gl = (t & (G - 1)) * L
            v = plsc.load_gather(raw_ref,
                                 [iota + gl, jnp.full((L,), c, jnp.int32)])
            dst_ref[c, pl.ds(gl, L)] = v

    def gather_descs(mv_i, it_i, p):
        d = [pltpu.make_async_copy(W_move.at[mv_i.at[p * 4 + m]],
                                   rows.at[m], sem_ga)
             for m in range(4)]
        d += [pltpu.make_async_copy(W_item.at[it_i.at[p * 2 + j]],
                                    rows.at[4 + j], sem_ga)
              for j in range(2)]
        return d

    def lookup(tab, idx_ref, idx_rows, n_d, tile_base, scale):
        for g in range(G):
            regs = [idx_ref[j, pl.ds(g * L, L)] for j in idx_rows]

            @plsc.parallel_loop(0, n_d, unroll=4)
            def body(d, regs=regs, g=g):
                col = jnp.full((L,), d, jnp.int32)
                v = None
                for r in regs:
                    x = plsc.load_gather(tab, [r, col])
                    v = x if v is None else v + x
                if scale != 1.0:
                    v = v * scale
                tile[tile_base + d, pl.ds(g * L, L)] = v

    def transpose_moves_items(sep_moves, mv_base, it_base):
        mv_rows = [[mv_base + m * 32 + h * L + iota for h in range(2)]
                   for m in range(4)]
        it_rows = [it_base + h * L + iota for h in range(2)]

        @plsc.parallel_loop(0, BC, unroll=4)
        def body(b):
            col = jnp.full((L,), b, jnp.int32)
            for h in range(2):
                sl = pl.ds(h * L, L)
                if sep_moves:
                    for m in range(4):
                        plsc.store_scatter(tile, [mv_rows[m][h], col],
                                           rows[m, b, sl])
                else:
                    v = rows[0, b, sl]
                    for m in range(1, 4):
                        v = v + rows[m, b, sl]
                    plsc.store_scatter(tile, [mv_rows[0][h], col], v * 0.25)
                vi = (rows[4, b, sl] + rows[5, b, sl]) * 0.5
                plsc.store_scatter(tile, [it_rows[h], col], vi)

    def scale_rows(tile_base, n_d, scale):
        @plsc.parallel_loop(0, G * n_d, unroll=4)
        def body(t):
            gl = (t & (G - 1)) * L
            d = t >> 3
            sl = pl.ds(gl, L)
            tile[tile_base + d, sl] = tile[tile_base + d, sl] * scale

    def out_desc(n_rows, row0, bsl):
        return pltpu.make_async_copy(tile.at[pl.ds(0, n_rows), pl.ds(0, BC)],
                                     out.at[pl.ds(row0, n_rows), bsl],
                                     sem_out)

    def stage(p, bsl, own):
        if own:
            mv_i, it_i, ab_i, ty_i, st_i = mv_c, it_c, ab_c, ty_c, st_c
            st_src, bo_src = stats_own, boosts_own
            ab_b, ty_b, stat_b, sa_b, bo_b = 160, 176, 192, 200, 207
            n_rows = OWN_PER
            row0 = OWN_BASE + p * OWN_PER
        else:
            mv_i, it_i, ab_i, ty_i, st_i = op_mv, op_it, op_ab, op_ty, op_st
            st_src, bo_src = stats_opp, boosts_opp
            ab_b, ty_b, stat_b, sa_b, bo_b = 64, 80, 96, 104, 111
            n_rows = OPP_PER
            row0 = OPP_BASE + p * OPP_PER

        g_descs = gather_descs(mv_i, it_i, p)
        for dsc in g_descs:
            dsc.start()

        @pl.when(p > 0)
        def _():
            out_desc(n_rows, row0, bsl).wait()

        stat_descs = [
            pltpu.make_async_copy(st_src.at[p, :, bsl],
                                  tile.at[pl.ds(sa_b, 7), pl.ds(0, BC)],
                                  sem_st),
            pltpu.make_async_copy(bo_src.at[p, :, bsl],
                                  tile.at[pl.ds(bo_b, 7), pl.ds(0, BC)],
                                  sem_st),
        ]
        for dsc in stat_descs:
            dsc.start()

        lookup(tab_ab, ab_i, [p * 2, p * 2 + 1], ABILITY_DIM, ab_b, 0.5)
        lookup(tab_ty, ty_i, [p * 2, p * 2 + 1], TYPE_DIM, ty_b, 1.0)
        lookup(tab_st, st_i, [p], STATUS_DIM, stat_b, 1.0)

        for dsc in g_descs:
            dsc.wait()
        if own:
            transpose_moves_items(True, 0, 128)
        else:
            transpose_moves_items(False, 0, 32)

        for dsc in stat_descs:
            dsc.wait()
        scale_rows(bo_b, 7, 1.0 / 6.0)

        out_desc(n_rows, row0, bsl).start()

    def raw_descs(side_refs, bsl):
        mv, it, ab, ty, st = side_refs
        return [
            pltpu.make_async_copy(mv.at[bsl], raw24, sem_idx),
            pltpu.make_async_copy(it.at[bsl], raw12a, sem_idx),
            pltpu.make_async_copy(ab.at[bsl], raw12b, sem_idx),
            pltpu.make_async_copy(ty.at[bsl], raw12c, sem_idx),
            pltpu.make_async_copy(st.at[bsl], raw6, sem_idx),
        ]

    def compact_side(dsts):
        deinterleave(raw24, 24, dsts[0])
        deinterleave(raw12a, 12, dsts[1])
        deinterleave(raw12b, 12, dsts[2])
        deinterleave(raw12c, 12, dsts[3])
        deinterleave(raw6, 6, dsts[4])

    def sub_body(sub, _):
        b0 = wid * BW + sub * BC
        bsl = pl.ds(b0, BC)

        own_raw = raw_descs((rmv_own, rit_own, rab_own, rty_own, rst_own),
                            bsl)
        head_raw = [
            pltpu.make_async_copy(rfld.at[bsl], rawh1, sem_idx),
            pltpu.make_async_copy(rsd.at[bsl], rawh2, sem_idx),
            pltpu.make_async_copy(rosd.at[bsl], rawh3, sem_idx),
            pltpu.make_async_copy(rdyn.at[bsl], rawh4, sem_idx),
        ]
        for dsc in own_raw + head_raw:
            dsc.start()
        for dsc in own_raw + head_raw:
            dsc.wait()

        compact_side((mv_c, it_c, ab_c, ty_c, st_c))

        opp_raw = raw_descs((rmv_opp, rit_opp, rab_opp, rty_opp, rst_opp),
                            bsl)
        for dsc in opp_raw:
            dsc.start()

        deinterleave(rawh1, 3, hd_i.at[pl.ds(0, 3)])
        deinterleave(rawh2, 4, hd_i.at[pl.ds(3, 4)])
        deinterleave(rawh3, 4, hd_i.at[pl.ds(7, 4)])
        for c, row in ((0, 16), (1, 17), (2, 34), (3, 35)):
            for g in range(G):
                tile[row, pl.ds(g * L, L)] = plsc.load_gather(
                    rawh4, [iota + g * L, jnp.full((L,), c, jnp.int32)])

        lookup(tab_fl, hd_i, [0, 1, 2], FIELD_DIM, 0, 1.0)
        lookup(tab_sd, hd_i, [3, 4, 5, 6], SIDE_DIM, 18, 1.0)
        lookup(tab_sd, hd_i, [7, 8, 9, 10], SIDE_DIM, 36, 1.0)
        hm_descs = [
            pltpu.make_async_copy(tile.at[pl.ds(0, 34), pl.ds(0, BC)],
                                  out.at[pl.ds(0, 34), bsl], sem_out),
            pltpu.make_async_copy(tile.at[pl.ds(34, 18), pl.ds(0, BC)],
                                  out.at[pl.ds(MID_BASE, 18), bsl], sem_out),
        ]
        for dsc in hm_descs:
            dsc.start()

        for dsc in opp_raw:
            dsc.wait()
        compact_side((op_mv, op_it, op_ab, op_ty, op_st))

        for dsc in hm_descs:
            dsc.wait()

        def own_body(p, _):
            stage(p, bsl, True)
            return 0
        lax.fori_loop(0, 6, own_body, 0)
        out_desc(OWN_PER, OWN_BASE + 5 * OWN_PER, bsl).wait()

        def opp_body(p, _):
            stage(p, bsl, False)
            return 0
        lax.fori_loop(0, 6, opp_body, 0)
        out_desc(OPP_PER, OPP_BASE + 5 * OPP_PER, bsl).wait()
        return 0

    lax.fori_loop(0, NSUB, sub_body, 0)


@jax.jit
def kernel(own_move_idx, own_item_idx, own_ability_idx, own_type_idx,
           own_status_idx, opp_move_idx, opp_item_idx, opp_ability_idx,
           opp_type_idx, opp_status_idx, field_attrib_idx, side_attrib_idx,
           opp_side_attrib_idx, own_stats, own_boosts, opp_stats, opp_boosts,
           dyn_flags, W_move, W_item, W_ability, W_side, W_field, W_type,
           W_status):
    def podd(a):
        a = a.reshape(B, -1).astype(jnp.int32)
        if a.shape[1] % 2 == 0:
            a = jnp.pad(a, ((0, 0), (0, 1)))
        return a

    f3 = lambda a: jnp.transpose(a, (1, 2, 0))
    dyn_p = jnp.pad(dyn_flags, ((0, 0), (0, 1)))
    mesh = plsc.VectorSubcoreMesh(core_axis_name="c", subcore_axis_name="s",
                                  num_cores=NC, num_subcores=NS)
    run = pl.kernel(
        _sc_encode,
        out_type=jax.ShapeDtypeStruct((D_OUT, B), jnp.float32),
        mesh=mesh,
        scratch_types=[
            pltpu.VMEM((1000, ABILITY_DIM + 1), jnp.float32),
            pltpu.VMEM((64, SIDE_DIM + 1), jnp.float32),
            pltpu.VMEM((64, FIELD_DIM + 1), jnp.float32),
            pltpu.VMEM((32, TYPE_DIM + 1), jnp.float32),
            pltpu.VMEM((16, STATUS_DIM + 1), jnp.float32),
            pltpu.VMEM((6, BC, 32), jnp.float32),
            pltpu.VMEM((24, BC), jnp.int32),
            pltpu.VMEM((12, BC), jnp.int32),
            pltpu.VMEM((12, BC), jnp.int32),
            pltpu.VMEM((12, BC), jnp.int32),
            pltpu.VMEM((6, BC), jnp.int32),
            pltpu.VMEM((24, BC), jnp.int32),
            pltpu.VMEM((12, BC), jnp.int32),
            pltpu.VMEM((12, BC), jnp.int32),
            pltpu.VMEM((12, BC), jnp.int32),
            pltpu.VMEM((6, BC), jnp.int32),
            pltpu.VMEM((12, BC), jnp.int32),
            pltpu.VMEM((BC, 25), jnp.int32),
            pltpu.VMEM((BC, 13), jnp.int32),
            pltpu.VMEM((BC, 13), jnp.int32),
            pltpu.VMEM((BC, 13), jnp.int32),
            pltpu.VMEM((BC, 7), jnp.int32),
            pltpu.VMEM((BC, 3), jnp.int32),
            pltpu.VMEM((BC, 5), jnp.int32),
            pltpu.VMEM((BC, 5), jnp.int32),
            pltpu.VMEM((BC, 5), jnp.float32),
            pltpu.VMEM((OWN_PER, BC + 1), jnp.float32),
            pltpu.SemaphoreType.DMA,
            pltpu.SemaphoreType.DMA,
            pltpu.SemaphoreType.DMA,
            pltpu.SemaphoreType.DMA,
            pltpu.SemaphoreType.DMA,
        ],
        compiler_params=pltpu.CompilerParams(use_tc_tiling_on_sc=False,
                                             needs_layout_passes=False),
    )
    return run(
        podd(own_move_idx), podd(own_item_idx), podd(own_ability_idx),
        podd(own_type_idx), podd(own_status_idx),
        podd(opp_move_idx), podd(opp_item_idx), podd(opp_ability_idx),
        podd(opp_type_idx), podd(opp_status_idx),
        podd(field_attrib_idx), podd(side_attrib_idx),
        podd(opp_side_attrib_idx),
        f3(own_stats), f3(own_boosts), f3(opp_stats), f3(opp_boosts),
        dyn_p,
        W_move, W_item, W_ability, W_side, W_field, W_type, W_status)


# speedup vs baseline: 14.3725x; 1.0220x over previous
import jax
import jax.numpy as jnp
from jax import lax
from jax.experimental import pallas as pl
from jax.experimental.pallas import tpu as pltpu
from jax.experimental.pallas import tpu_sc as plsc

B = 16384
NC, NS, L = 2, 16, 16
NW = NC * NS
BW = B // NW
BC = 128
G = BC // L
NSUB = BW // BC

MOVE_DIM, ITEM_DIM, ABILITY_DIM = 32, 32, 16
SIDE_DIM, FIELD_DIM, TYPE_DIM, STATUS_DIM = 16, 16, 16, 8

OWN_BASE = 34
OWN_PER = 214
MID_BASE = OWN_BASE + 6 * OWN_PER
OPP_BASE = MID_BASE + 18
OPP_PER = 118
D_OUT = OPP_BASE + 6 * OPP_PER


def _sc_encode(rmv_own, rit_own, rab_own, rty_own, rst_own,
               rmv_opp, rit_opp, rab_opp, rty_opp, rst_opp,
               rfld, rsd, rosd, stats_own, boosts_own, stats_opp, boosts_opp,
               rdyn, W_move, W_item, W_ability, W_side, W_field, W_type,
               W_status, out,
               tab_ab, tab_sd, tab_fl, tab_ty, tab_st,
               rows, mv_c, it_c, ab_c, ty_c, st_c, op_mv, op_it, op_ab,
               op_ty, op_st, hd_i,
               raw24, raw12a, raw12b, raw12c, raw6,
               rawh1, rawh2, rawh3, rawh4, tile,
               sem_ga, sem_gb, sem_idx, sem_out, sem_st):
    cid = lax.axis_index("c")
    sid = lax.axis_index("s")
    wid = sid * NC + cid

    iota = lax.iota(jnp.int32, L)

    pltpu.sync_copy(W_ability, tab_ab.at[:, pl.ds(0, ABILITY_DIM)])
    pltpu.sync_copy(W_side, tab_sd.at[:, pl.ds(0, SIDE_DIM)])
    pltpu.sync_copy(W_field, tab_fl.at[:, pl.ds(0, FIELD_DIM)])
    pltpu.sync_copy(W_type, tab_ty.at[:, pl.ds(0, TYPE_DIM)])
    pltpu.sync_copy(W_status, tab_st.at[:, pl.ds(0, STATUS_DIM)])

    def deinterleave(raw_ref, n_c, dst_ref):
        @plsc.parallel_loop(0, n_c * G, unroll=4)
        def body(t):
            c = t >> 3
            gl = (t & (G - 1)) * L
            v = plsc.load_gather(raw_ref,
                                 [iota + gl, jnp.full((L,), c, jnp.int32)])
            dst_ref[c, pl.ds(gl, L)] = v

    def gather_descs(mv_i, it_i, p):
        d = [pltpu.make_async_copy(W_move.at[mv_i.at[p * 4 + m]],
                                   rows.at[m], sem_ga)
             for m in range(4)]
        d += [pltpu.make_async_copy(W_item.at[it_i.at[p * 2 + j]],
                                    rows.at[4 + j], sem_ga)
              for j in range(2)]
        return d

    def lookup(tab, idx_ref, idx_rows, n_d, tile_base, scale):
        for g in range(G):
            regs = [idx_ref[j, pl.ds(g * L, L)] for j in idx_rows]

            @plsc.parallel_loop(0, n_d, unroll=4)
            def body(d, regs=regs, g=g):
                col = jnp.full((L,), d, jnp.int32)
                v = None
                for r in regs:
                    x = plsc.load_gather(tab, [r, col])
                    v = x if v is None else v + x
                if scale != 1.0:
                    v = v * scale
                tile[tile_base + d, pl.ds(g * L, L)] = v

    def transpose_moves_items(sep_moves, mv_base, it_base):
        mv_rows = [[mv_base + m * 32 + h * L + iota for h in range(2)]
                   for m in range(4)]
        it_rows = [it_base + h * L + iota for h in range(2)]

        @plsc.parallel_loop(0, BC, unroll=4)
        def body(b):
            col = jnp.full((L,), b, jnp.int32)
            for h in range(2):
                sl = pl.ds(h * L, L)
                if sep_moves:
                    for m in range(4):
                        plsc.store_scatter(tile, [mv_rows[m][h], col],
                                           rows[m, b, sl])
                else:
                    v = rows[0, b, sl]
                    for m in range(1, 4):
                        v = v + rows[m, b, sl]
                    plsc.store_scatter(tile, [mv_rows[0][h], col], v * 0.25)
                vi = (rows[4, b, sl] + rows[5, b, sl]) * 0.5
                plsc.store_scatter(tile, [it_rows[h], col], vi)

    def scale_rows(tile_base, n_d, scale):
        @plsc.parallel_loop(0, G * n_d, unroll=4)
        def body(t):
            gl = (t & (G - 1)) * L
            d = t >> 3
            sl = pl.ds(gl, L)
            tile[tile_base + d, sl] = tile[tile_base + d, sl] * scale

    def out_desc(n_rows, row0, bsl):
        return pltpu.make_async_copy(tile.at[pl.ds(0, n_rows), pl.ds(0, BC)],
                                     out.at[pl.ds(row0, n_rows), bsl],
                                     sem_out)

    def stage(p, bsl, own):
        if own:
            mv_i, it_i, ab_i, ty_i, st_i = mv_c, it_c, ab_c, ty_c, st_c
            st_src, bo_src = stats_own, boosts_own
            ab_b, ty_b, stat_b, sa_b, bo_b = 160, 176, 192, 200, 207
            n_rows = OWN_PER
            row0 = OWN_BASE + p * OWN_PER
        else:
            mv_i, it_i, ab_i, ty_i, st_i = op_mv, op_it, op_ab, op_ty, op_st
            st_src, bo_src = stats_opp, boosts_opp
            ab_b, ty_b, stat_b, sa_b, bo_b = 64, 80, 96, 104, 111
            n_rows = OPP_PER
            row0 = OPP_BASE + p * OPP_PER

        g_descs = gather_descs(mv_i, it_i, p)
        for dsc in g_descs:
            dsc.start()

        @pl.when(p > 0)
        def _():
            out_desc(n_rows, row0, bsl).wait()

        stat_descs = [
            pltpu.make_async_copy(st_src.at[p, :, bsl],
                                  tile.at[pl.ds(sa_b, 7), pl.ds(0, BC)],
                                  sem_st),
            pltpu.make_async_copy(bo_src.at[p, :, bsl],
                                  tile.at[pl.ds(bo_b, 7), pl.ds(0, BC)],
                                  sem_st),
        ]
        for dsc in stat_descs:
            dsc.start()

        lookup(tab_ab, ab_i, [p * 2, p * 2 + 1], ABILITY_DIM, ab_b, 0.5)
        lookup(tab_ty, ty_i, [p * 2, p * 2 + 1], TYPE_DIM, ty_b, 1.0)
        lookup(tab_st, st_i, [p], STATUS_DIM, stat_b, 1.0)

        for dsc in g_descs:
            dsc.wait()
        if own:
            transpose_moves_items(True, 0, 128)
        else:
            transpose_moves_items(False, 0, 32)

        for dsc in stat_descs:
            dsc.wait()
        scale_rows(bo_b, 7, 1.0 / 6.0)

        out_desc(n_rows, row0, bsl).start()

    def raw_descs(side_refs, bsl):
        mv, it, ab, ty, st = side_refs
        return [
            pltpu.make_async_copy(mv.at[bsl], raw24, sem_idx),
            pltpu.make_async_copy(it.at[bsl], raw12a, sem_idx),
            pltpu.make_async_copy(ab.at[bsl], raw12b, sem_idx),
            pltpu.make_async_copy(ty.at[bsl], raw12c, sem_idx),
            pltpu.make_async_copy(st.at[bsl], raw6, sem_idx),
        ]

    def compact_side(dsts):
        deinterleave(raw24, 24, dsts[0])
        deinterleave(raw12a, 12, dsts[1])
        deinterleave(raw12b, 12, dsts[2])
        deinterleave(raw12c, 12, dsts[3])
        deinterleave(raw6, 6, dsts[4])

    def sub_body(sub, _):
        b0 = wid * BW + sub * BC
        bsl = pl.ds(b0, BC)

        own_raw = raw_descs((rmv_own, rit_own, rab_own, rty_own, rst_own),
                            bsl)
        head_raw = [
            pltpu.make_async_copy(rfld.at[bsl], rawh1, sem_idx),
            pltpu.make_async_copy(rsd.at[bsl], rawh2, sem_idx),
            pltpu.make_async_copy(rosd.at[bsl], rawh3, sem_idx),
            pltpu.make_async_copy(rdyn.at[bsl], rawh4, sem_idx),
        ]
        for dsc in own_raw + head_raw:
            dsc.start()
        for dsc in own_raw + head_raw:
            dsc.wait()

        compact_side((mv_c, it_c, ab_c, ty_c, st_c))

        opp_raw = raw_descs((rmv_opp, rit_opp, rab_opp, rty_opp, rst_opp),
                            bsl)
        for dsc in opp_raw:
            dsc.start()

        deinterleave(rawh1, 3, hd_i.at[pl.ds(0, 3)])
        deinterleave(rawh2, 4, hd_i.at[pl.ds(3, 4)])
        deinterleave(rawh3, 4, hd_i.at[pl.ds(7, 4)])
        for c, row in ((0, 16), (1, 17), (2, 34), (3, 35)):
            for g in range(G):
                tile[row, pl.ds(g * L, L)] = plsc.load_gather(
                    rawh4, [iota + g * L, jnp.full((L,), c, jnp.int32)])

        lookup(tab_fl, hd_i, [0, 1, 2], FIELD_DIM, 0, 1.0)
        lookup(tab_sd, hd_i, [3, 4, 5, 6], SIDE_DIM, 18, 1.0)
        lookup(tab_sd, hd_i, [7, 8, 9, 10], SIDE_DIM, 36, 1.0)
        hm_descs = [
            pltpu.make_async_copy(tile.at[pl.ds(0, 34), pl.ds(0, BC)],
                                  out.at[pl.ds(0, 34), bsl], sem_out),
            pltpu.make_async_copy(tile.at[pl.ds(34, 18), pl.ds(0, BC)],
                                  out.at[pl.ds(MID_BASE, 18), bsl], sem_out),
        ]
        for dsc in hm_descs:
            dsc.start()

        for dsc in opp_raw:
            dsc.wait()
        compact_side((op_mv, op_it, op_ab, op_ty, op_st))

        for dsc in hm_descs:
            dsc.wait()

        def own_body(p, _):
            stage(p, bsl, True)
            return 0
        lax.fori_loop(0, 6, own_body, 0)
        out_desc(OWN_PER, OWN_BASE + 5 * OWN_PER, bsl).wait()

        def opp_body(p, _):
            stage(p, bsl, False)
            return 0
        lax.fori_loop(0, 6, opp_body, 0)
        out_desc(OPP_PER, OPP_BASE + 5 * OPP_PER, bsl).wait()
        return 0

    lax.fori_loop(0, NSUB, sub_body, 0)


@jax.jit
def kernel(own_move_idx, own_item_idx, own_ability_idx, own_type_idx,
           own_status_idx, opp_move_idx, opp_item_idx, opp_ability_idx,
           opp_type_idx, opp_status_idx, field_attrib_idx, side_attrib_idx,
           opp_side_attrib_idx, own_stats, own_boosts, opp_stats, opp_boosts,
           dyn_flags, W_move, W_item, W_ability, W_side, W_field, W_type,
           W_status):
    r2 = lambda a: a.reshape(B, -1).astype(jnp.int32)
    f3 = lambda a: jnp.transpose(a, (1, 2, 0))
    mesh = plsc.VectorSubcoreMesh(core_axis_name="c", subcore_axis_name="s",
                                  num_cores=NC, num_subcores=NS)
    run = pl.kernel(
        _sc_encode,
        out_type=jax.ShapeDtypeStruct((D_OUT, B), jnp.float32),
        mesh=mesh,
        scratch_types=[
            pltpu.VMEM((1000, ABILITY_DIM + 1), jnp.float32),
            pltpu.VMEM((64, SIDE_DIM + 1), jnp.float32),
            pltpu.VMEM((64, FIELD_DIM + 1), jnp.float32),
            pltpu.VMEM((32, TYPE_DIM + 1), jnp.float32),
            pltpu.VMEM((16, STATUS_DIM + 1), jnp.float32),
            pltpu.VMEM((6, BC, 32), jnp.float32),
            pltpu.VMEM((24, BC), jnp.int32),
            pltpu.VMEM((12, BC), jnp.int32),
            pltpu.VMEM((12, BC), jnp.int32),
            pltpu.VMEM((12, BC), jnp.int32),
            pltpu.VMEM((6, BC), jnp.int32),
            pltpu.VMEM((24, BC), jnp.int32),
            pltpu.VMEM((12, BC), jnp.int32),
            pltpu.VMEM((12, BC), jnp.int32),
            pltpu.VMEM((12, BC), jnp.int32),
            pltpu.VMEM((6, BC), jnp.int32),
            pltpu.VMEM((12, BC), jnp.int32),
            pltpu.VMEM((BC, 24), jnp.int32),
            pltpu.VMEM((BC, 12), jnp.int32),
            pltpu.VMEM((BC, 12), jnp.int32),
            pltpu.VMEM((BC, 12), jnp.int32),
            pltpu.VMEM((BC, 6), jnp.int32),
            pltpu.VMEM((BC, 3), jnp.int32),
            pltpu.VMEM((BC, 4), jnp.int32),
            pltpu.VMEM((BC, 4), jnp.int32),
            pltpu.VMEM((BC, 4), jnp.float32),
            pltpu.VMEM((OWN_PER, BC + 1), jnp.float32),
            pltpu.SemaphoreType.DMA,
            pltpu.SemaphoreType.DMA,
            pltpu.SemaphoreType.DMA,
            pltpu.SemaphoreType.DMA,
            pltpu.SemaphoreType.DMA,
        ],
        compiler_params=pltpu.CompilerParams(use_tc_tiling_on_sc=False,
                                             needs_layout_passes=False),
    )
    return run(
        r2(own_move_idx), r2(own_item_idx), r2(own_ability_idx),
        r2(own_type_idx), r2(own_status_idx),
        r2(opp_move_idx), r2(opp_item_idx), r2(opp_ability_idx),
        r2(opp_type_idx), r2(opp_status_idx),
        r2(field_attrib_idx), r2(side_attrib_idx), r2(opp_side_attrib_idx),
        f3(own_stats), f3(own_boosts), f3(opp_stats), f3(opp_boosts),
        dyn_flags,
        W_move, W_item, W_ability, W_side, W_field, W_type, W_status)


# speedup vs baseline: 20.8478x; 1.4505x over previous
import jax
import jax.numpy as jnp
from jax import lax
from jax.experimental import pallas as pl
from jax.experimental.pallas import tpu as pltpu
from jax.experimental.pallas import tpu_sc as plsc

B = 16384
NC, NS, L = 2, 16, 16
NW = NC * NS
BW = B // NW
BC = 128
G = BC // L
NSUB = BW // BC

MOVE_DIM, ITEM_DIM, ABILITY_DIM = 32, 32, 16
SIDE_DIM, FIELD_DIM, TYPE_DIM, STATUS_DIM = 16, 16, 16, 8

OWN_BASE = 34
OWN_PER = 214
MID_BASE = OWN_BASE + 6 * OWN_PER
OPP_BASE = MID_BASE + 18
OPP_PER = 118
D_OUT = OPP_BASE + 6 * OPP_PER


def _sc_encode(mv_own, it_own, ab_own, ty_own, st_own,
               mv_opp, it_opp, ab_opp, ty_opp, st_opp,
               fld, sd, osd, stats_own, boosts_own, stats_opp, boosts_opp,
               dyn, W_move, W_item, W_ability, W_side, W_field, W_type,
               W_status, out,
               tab_ab, tab_sd, tab_fl, tab_ty, tab_st,
               rows, mvo_i, ito_i, abo_i, tyo_i, sto_i,
               mvp_i, itp_i, abp_i, typ_i, stp_i, hd_i, tile,
               sem_ga, sem_gb, sem_idx, sem_out, sem_st):
    cid = lax.axis_index("c")
    sid = lax.axis_index("s")
    wid = sid * NC + cid

    iota = lax.iota(jnp.int32, L)

    pltpu.sync_copy(W_ability, tab_ab.at[:, pl.ds(0, ABILITY_DIM)])
    pltpu.sync_copy(W_side, tab_sd.at[:, pl.ds(0, SIDE_DIM)])
    pltpu.sync_copy(W_field, tab_fl.at[:, pl.ds(0, FIELD_DIM)])
    pltpu.sync_copy(W_type, tab_ty.at[:, pl.ds(0, TYPE_DIM)])
    pltpu.sync_copy(W_status, tab_st.at[:, pl.ds(0, STATUS_DIM)])

    def gather_descs(mv_i, it_i, p, par, sem):
        d = [pltpu.make_async_copy(W_move.at[mv_i.at[p, m]],
                                   rows.at[par, m], sem)
             for m in range(4)]
        d += [pltpu.make_async_copy(W_item.at[it_i.at[p, j]],
                                    rows.at[par, 4 + j], sem)
              for j in range(2)]
        return d

    def start_gathers(mv_i, it_i, p, par, sem):
        for dsc in gather_descs(mv_i, it_i, p, par, sem):
            dsc.start()

    def drain_gathers(sem):
        for dsc in gather_descs(mvo_i, ito_i, 0, 0, sem):
            dsc.wait()

    def lookup(tab, idx_ref, idx_rows, n_d, tile_base, scale):
        for g in range(G):
            regs = [idx_ref[j, pl.ds(g * L, L)] for j in idx_rows]

            @plsc.parallel_loop(0, n_d, unroll=4)
            def body(d, regs=regs, g=g):
                col = jnp.full((L,), d, jnp.int32)
                v = None
                for r in regs:
                    x = plsc.load_gather(tab, [r, col])
                    v = x if v is None else v + x
                if scale != 1.0:
                    v = v * scale
                tile[tile_base + d, pl.ds(g * L, L)] = v

    def transpose_moves_items(par, sep_moves, mv_base, it_base):
        mv_rows = [[mv_base + m * 32 + h * L + iota for h in range(2)]
                   for m in range(4)]
        it_rows = [it_base + h * L + iota for h in range(2)]

        @plsc.parallel_loop(0, BC, unroll=8)
        def body(b):
            col = jnp.full((L,), b, jnp.int32)
            for h in range(2):
                sl = pl.ds(h * L, L)
                if sep_moves:
                    for m in range(4):
                        plsc.store_scatter(tile, [mv_rows[m][h], col],
                                           rows[par, m, b, sl])
                else:
                    v = rows[par, 0, b, sl]
                    for m in range(1, 4):
                        v = v + rows[par, m, b, sl]
                    plsc.store_scatter(tile, [mv_rows[0][h], col], v * 0.25)
                vi = (rows[par, 4, b, sl] + rows[par, 5, b, sl]) * 0.5
                plsc.store_scatter(tile, [it_rows[h], col], vi)

    def scale_rows(tile_base, n_d, scale):
        @plsc.parallel_loop(0, G * n_d, unroll=4)
        def body(t):
            gl = (t & (G - 1)) * L
            d = t >> 3
            sl = pl.ds(gl, L)
            tile[tile_base + d, sl] = tile[tile_base + d, sl] * scale

    def out_desc(n_rows, row0, bsl):
        return pltpu.make_async_copy(tile.at[pl.ds(0, n_rows), pl.ds(0, BC)],
                                     out.at[pl.ds(row0, n_rows), bsl],
                                     sem_out)

    def stage(p, bsl, own, own_last):
        if own:
            mv_i, it_i, ab_i, ty_i, st_i = mvo_i, ito_i, abo_i, tyo_i, sto_i
            st_src, bo_src = stats_own, boosts_own
            ab_b, ty_b, stat_b, sa_b, bo_b = 160, 176, 192, 200, 207
            n_rows = OWN_PER
            row0 = OWN_BASE + p * OWN_PER
        else:
            mv_i, it_i, ab_i, ty_i, st_i = mvp_i, itp_i, abp_i, typ_i, stp_i
            st_src, bo_src = stats_opp, boosts_opp
            ab_b, ty_b, stat_b, sa_b, bo_b = 64, 80, 96, 104, 111
            n_rows = OPP_PER
            row0 = OPP_BASE + p * OPP_PER

        par = p & 1

        @pl.when((p < 5) & (par == 0))
        def _():
            start_gathers(mv_i, it_i, p + 1, 1 - par, sem_gb)

        @pl.when((p < 5) & (par == 1))
        def _():
            start_gathers(mv_i, it_i, p + 1, 1 - par, sem_ga)

        @pl.when(own_last & (p == 5))
        def _():
            start_gathers(mvp_i, itp_i, 0, 0, sem_ga)

        @pl.when(p > 0)
        def _():
            out_desc(n_rows, row0, bsl).wait()

        stat_descs = [
            pltpu.make_async_copy(st_src.at[p, :, bsl],
                                  tile.at[pl.ds(sa_b, 7), pl.ds(0, BC)], sem_st),
            pltpu.make_async_copy(bo_src.at[p, :, bsl],
                                  tile.at[pl.ds(bo_b, 7), pl.ds(0, BC)], sem_st),
        ]
        for dsc in stat_descs:
            dsc.start()

        lookup(tab_ab, ab_i.at[p], [0, 1], ABILITY_DIM, ab_b, 0.5)
        lookup(tab_ty, ty_i.at[p], [0, 1], TYPE_DIM, ty_b, 1.0)
        lookup(tab_st, st_i, [p], STATUS_DIM, stat_b, 1.0)

        @pl.when(par == 0)
        def _():
            drain_gathers(sem_ga)

        @pl.when(par == 1)
        def _():
            drain_gathers(sem_gb)

        if own:
            transpose_moves_items(par, True, 0, 128)
        else:
            transpose_moves_items(par, False, 0, 32)

        for dsc in stat_descs:
            dsc.wait()
        scale_rows(bo_b, 7, 1.0 / 6.0)

        out_desc(n_rows, row0, bsl).start()

    def sub_body(sub, _):
        b0 = wid * BW + sub * BC
        bsl = pl.ds(b0, BC)

        idx_descs = [
            pltpu.make_async_copy(mv_own.at[:, :, bsl], mvo_i, sem_idx),
            pltpu.make_async_copy(it_own.at[:, :, bsl], ito_i, sem_idx),
            pltpu.make_async_copy(ab_own.at[:, :, bsl], abo_i, sem_idx),
            pltpu.make_async_copy(ty_own.at[:, :, bsl], tyo_i, sem_idx),
            pltpu.make_async_copy(st_own.at[:, bsl], sto_i, sem_idx),
            pltpu.make_async_copy(mv_opp.at[:, :, bsl], mvp_i, sem_idx),
            pltpu.make_async_copy(it_opp.at[:, :, bsl], itp_i, sem_idx),
            pltpu.make_async_copy(ab_opp.at[:, :, bsl], abp_i, sem_idx),
            pltpu.make_async_copy(ty_opp.at[:, :, bsl], typ_i, sem_idx),
            pltpu.make_async_copy(st_opp.at[:, bsl], stp_i, sem_idx),
            pltpu.make_async_copy(fld.at[:, bsl], hd_i.at[pl.ds(0, 3)],
                                  sem_idx),
            pltpu.make_async_copy(sd.at[:, bsl], hd_i.at[pl.ds(3, 4)],
                                  sem_idx),
            pltpu.make_async_copy(osd.at[:, bsl], hd_i.at[pl.ds(7, 4)],
                                  sem_idx),
        ]
        for dsc in idx_descs:
            dsc.start()
        for dsc in idx_descs:
            dsc.wait()

        lookup(tab_fl, hd_i, [0, 1, 2], FIELD_DIM, 0, 1.0)
        lookup(tab_sd, hd_i, [3, 4, 5, 6], SIDE_DIM, 18, 1.0)
        lookup(tab_sd, hd_i, [7, 8, 9, 10], SIDE_DIM, 36, 1.0)
        pltpu.sync_copy(dyn.at[pl.ds(0, 2), bsl], tile.at[pl.ds(16, 2), pl.ds(0, BC)])
        pltpu.sync_copy(dyn.at[pl.ds(2, 2), bsl], tile.at[pl.ds(34, 2), pl.ds(0, BC)])
        hm_descs = [
            pltpu.make_async_copy(tile.at[pl.ds(0, 34), pl.ds(0, BC)],
                                  out.at[pl.ds(0, 34), bsl], sem_out),
            pltpu.make_async_copy(tile.at[pl.ds(34, 18), pl.ds(0, BC)],
                                  out.at[pl.ds(MID_BASE, 18), bsl], sem_out),
        ]
        for dsc in hm_descs:
            dsc.start()
        for dsc in hm_descs:
            dsc.wait()

        start_gathers(mvo_i, ito_i, 0, 0, sem_ga)

        def own_body(p, _):
            stage(p, bsl, True, jnp.bool_(True))
            return 0
        lax.fori_loop(0, 6, own_body, 0)
        out_desc(OWN_PER, OWN_BASE + 5 * OWN_PER, bsl).wait()

        def opp_body(p, _):
            stage(p, bsl, False, jnp.bool_(False))
            return 0
        lax.fori_loop(0, 6, opp_body, 0)
        out_desc(OPP_PER, OPP_BASE + 5 * OPP_PER, bsl).wait()
        return 0

    lax.fori_loop(0, NSUB, sub_body, 0)


@jax.jit
def kernel(own_move_idx, own_item_idx, own_ability_idx, own_type_idx,
           own_status_idx, opp_move_idx, opp_item_idx, opp_ability_idx,
           opp_type_idx, opp_status_idx, field_attrib_idx, side_attrib_idx,
           opp_side_attrib_idx, own_stats, own_boosts, opp_stats, opp_boosts,
           dyn_flags, W_move, W_item, W_ability, W_side, W_field, W_type,
           W_status):
    t3 = lambda a: jnp.transpose(a, (1, 2, 0)).astype(jnp.int32)
    f3 = lambda a: jnp.transpose(a, (1, 2, 0))
    mesh = plsc.VectorSubcoreMesh(core_axis_name="c", subcore_axis_name="s",
                                  num_cores=NC, num_subcores=NS)
    run = pl.kernel(
        _sc_encode,
        out_type=jax.ShapeDtypeStruct((D_OUT, B), jnp.float32),
        mesh=mesh,
        scratch_types=[
            pltpu.VMEM((1000, ABILITY_DIM + 1), jnp.float32),
            pltpu.VMEM((64, SIDE_DIM + 1), jnp.float32),
            pltpu.VMEM((64, FIELD_DIM + 1), jnp.float32),
            pltpu.VMEM((32, TYPE_DIM + 1), jnp.float32),
            pltpu.VMEM((16, STATUS_DIM + 1), jnp.float32),
            pltpu.VMEM((2, 6, BC, 32), jnp.float32),
            pltpu.VMEM((6, 4, BC), jnp.int32),
            pltpu.VMEM((6, 2, BC), jnp.int32),
            pltpu.VMEM((6, 2, BC), jnp.int32),
            pltpu.VMEM((6, 2, BC), jnp.int32),
            pltpu.VMEM((6, BC), jnp.int32),
            pltpu.VMEM((6, 4, BC), jnp.int32),
            pltpu.VMEM((6, 2, BC), jnp.int32),
            pltpu.VMEM((6, 2, BC), jnp.int32),
            pltpu.VMEM((6, 2, BC), jnp.int32),
            pltpu.VMEM((6, BC), jnp.int32),
            pltpu.VMEM((12, BC), jnp.int32),
            pltpu.VMEM((OWN_PER, BC + 1), jnp.float32),
            pltpu.SemaphoreType.DMA,
            pltpu.SemaphoreType.DMA,
            pltpu.SemaphoreType.DMA,
            pltpu.SemaphoreType.DMA,
            pltpu.SemaphoreType.DMA,
        ],
        compiler_params=pltpu.CompilerParams(use_tc_tiling_on_sc=False,
                                             needs_layout_passes=False),
    )
    return run(
        t3(own_move_idx), t3(own_item_idx), t3(own_ability_idx),
        t3(own_type_idx), jnp.transpose(own_status_idx).astype(jnp.int32),
        t3(opp_move_idx), t3(opp_item_idx), t3(opp_ability_idx),
        t3(opp_type_idx), jnp.transpose(opp_status_idx).astype(jnp.int32),
        jnp.transpose(field_attrib_idx).astype(jnp.int32),
        jnp.transpose(side_attrib_idx).astype(jnp.int32),
        jnp.transpose(opp_side_attrib_idx).astype(jnp.int32),
        f3(own_stats), f3(own_boosts), f3(opp_stats), f3(opp_boosts),
        jnp.transpose(dyn_flags),
        W_move, W_item, W_ability, W_side, W_field, W_type, W_status)


# speedup vs baseline: 22.3995x; 1.0744x over previous
import jax
import jax.numpy as jnp
from jax import lax
from jax.experimental import pallas as pl
from jax.experimental.pallas import tpu as pltpu
from jax.experimental.pallas import tpu_sc as plsc

B = 16384
NC, NS, L = 2, 16, 16
NW = NC * NS
BW = B // NW
BC = 128
G = BC // L
NSUB = BW // BC

MOVE_DIM, ITEM_DIM, ABILITY_DIM = 32, 32, 16
SIDE_DIM, FIELD_DIM, TYPE_DIM, STATUS_DIM = 16, 16, 16, 8

OWN_BASE = 34
OWN_PER = 214
MID_BASE = OWN_BASE + 6 * OWN_PER
OPP_BASE = MID_BASE + 18
OPP_PER = 118
D_OUT = OPP_BASE + 6 * OPP_PER


def _sc_encode(mv_own, it_own, ab_own, ty_own, st_own,
               mv_opp, it_opp, ab_opp, ty_opp, st_opp,
               fld, sd, osd, stats_own, boosts_own, stats_opp, boosts_opp,
               dyn, W_move, W_item, W_ability, W_side, W_field, W_type,
               W_status, out,
               tab_ab, tab_sd, tab_fl, tab_ty, tab_st,
               rows, mvo_i, ito_i, abo_i, tyo_i, sto_i,
               mvp_i, itp_i, abp_i, typ_i, stp_i, hd_i, tile,
               sem_ga, sem_gb, sem_idx, sem_out, sem_st):
    cid = lax.axis_index("c")
    sid = lax.axis_index("s")
    wid = sid * NC + cid

    iota = lax.iota(jnp.int32, L)

    pltpu.sync_copy(W_ability, tab_ab.at[:, pl.ds(0, ABILITY_DIM)])
    pltpu.sync_copy(W_side, tab_sd.at[:, pl.ds(0, SIDE_DIM)])
    pltpu.sync_copy(W_field, tab_fl.at[:, pl.ds(0, FIELD_DIM)])
    pltpu.sync_copy(W_type, tab_ty.at[:, pl.ds(0, TYPE_DIM)])
    pltpu.sync_copy(W_status, tab_st.at[:, pl.ds(0, STATUS_DIM)])

    def gather_descs(mv_i, it_i, p, par, sem):
        d = [pltpu.make_async_copy(W_move.at[mv_i.at[p, m]],
                                   rows.at[par, m], sem)
             for m in range(4)]
        d += [pltpu.make_async_copy(W_item.at[it_i.at[p, j]],
                                    rows.at[par, 4 + j], sem)
              for j in range(2)]
        return d

    def start_gathers(mv_i, it_i, p, par, sem):
        for dsc in gather_descs(mv_i, it_i, p, par, sem):
            dsc.start()

    def drain_gathers(sem):
        for dsc in gather_descs(mvo_i, ito_i, 0, 0, sem):
            dsc.wait()

    def lookup(tab, idx_ref, idx_rows, n_d, tile_base, scale):
        for g in range(G):
            regs = [idx_ref[j, pl.ds(g * L, L)] for j in idx_rows]

            @plsc.parallel_loop(0, n_d, unroll=4)
            def body(d, regs=regs, g=g):
                col = jnp.full((L,), d, jnp.int32)
                v = None
                for r in regs:
                    x = plsc.load_gather(tab, [r, col])
                    v = x if v is None else v + x
                if scale != 1.0:
                    v = v * scale
                tile[tile_base + d, pl.ds(g * L, L)] = v

    def transpose_moves_items(par, sep_moves, mv_base, it_base):
        mv_rows = [[mv_base + m * 32 + h * L + iota for h in range(2)]
                   for m in range(4)]
        it_rows = [it_base + h * L + iota for h in range(2)]

        @plsc.parallel_loop(0, BC, unroll=4)
        def body(b):
            col = jnp.full((L,), b, jnp.int32)
            for h in range(2):
                sl = pl.ds(h * L, L)
                if sep_moves:
                    for m in range(4):
                        plsc.store_scatter(tile, [mv_rows[m][h], col],
                                           rows[par, m, b, sl])
                else:
                    v = rows[par, 0, b, sl]
                    for m in range(1, 4):
                        v = v + rows[par, m, b, sl]
                    plsc.store_scatter(tile, [mv_rows[0][h], col], v * 0.25)
                vi = (rows[par, 4, b, sl] + rows[par, 5, b, sl]) * 0.5
                plsc.store_scatter(tile, [it_rows[h], col], vi)

    def scale_rows(tile_base, n_d, scale):
        @plsc.parallel_loop(0, G * n_d, unroll=4)
        def body(t):
            gl = (t & (G - 1)) * L
            d = t >> 3
            sl = pl.ds(gl, L)
            tile[tile_base + d, sl] = tile[tile_base + d, sl] * scale

    def out_desc(n_rows, row0, bsl):
        return pltpu.make_async_copy(tile.at[pl.ds(0, n_rows), pl.ds(0, BC)],
                                     out.at[pl.ds(row0, n_rows), bsl],
                                     sem_out)

    def stage(p, bsl, own, own_last):
        if own:
            mv_i, it_i, ab_i, ty_i, st_i = mvo_i, ito_i, abo_i, tyo_i, sto_i
            st_src, bo_src = stats_own, boosts_own
            ab_b, ty_b, stat_b, sa_b, bo_b = 160, 176, 192, 200, 207
            n_rows = OWN_PER
            row0 = OWN_BASE + p * OWN_PER
        else:
            mv_i, it_i, ab_i, ty_i, st_i = mvp_i, itp_i, abp_i, typ_i, stp_i
            st_src, bo_src = stats_opp, boosts_opp
            ab_b, ty_b, stat_b, sa_b, bo_b = 64, 80, 96, 104, 111
            n_rows = OPP_PER
            row0 = OPP_BASE + p * OPP_PER

        par = p & 1

        @pl.when((p < 5) & (par == 0))
        def _():
            start_gathers(mv_i, it_i, p + 1, 1 - par, sem_gb)

        @pl.when((p < 5) & (par == 1))
        def _():
            start_gathers(mv_i, it_i, p + 1, 1 - par, sem_ga)

        @pl.when(own_last & (p == 5))
        def _():
            start_gathers(mvp_i, itp_i, 0, 0, sem_ga)

        @pl.when(p > 0)
        def _():
            out_desc(n_rows, row0, bsl).wait()

        stat_descs = [
            pltpu.make_async_copy(st_src.at[p, :, bsl],
                                  tile.at[pl.ds(sa_b, 7), pl.ds(0, BC)], sem_st),
            pltpu.make_async_copy(bo_src.at[p, :, bsl],
                                  tile.at[pl.ds(bo_b, 7), pl.ds(0, BC)], sem_st),
        ]
        for dsc in stat_descs:
            dsc.start()

        lookup(tab_ab, ab_i.at[p], [0, 1], ABILITY_DIM, ab_b, 0.5)
        lookup(tab_ty, ty_i.at[p], [0, 1], TYPE_DIM, ty_b, 1.0)
        lookup(tab_st, st_i, [p], STATUS_DIM, stat_b, 1.0)

        @pl.when(par == 0)
        def _():
            drain_gathers(sem_ga)

        @pl.when(par == 1)
        def _():
            drain_gathers(sem_gb)

        if own:
            transpose_moves_items(par, True, 0, 128)
        else:
            transpose_moves_items(par, False, 0, 32)

        for dsc in stat_descs:
            dsc.wait()
        scale_rows(bo_b, 7, 1.0 / 6.0)

        out_desc(n_rows, row0, bsl).start()

    def sub_body(sub, _):
        b0 = wid * BW + sub * BC
        bsl = pl.ds(b0, BC)

        idx_descs = [
            pltpu.make_async_copy(mv_own.at[:, :, bsl], mvo_i, sem_idx),
            pltpu.make_async_copy(it_own.at[:, :, bsl], ito_i, sem_idx),
            pltpu.make_async_copy(ab_own.at[:, :, bsl], abo_i, sem_idx),
            pltpu.make_async_copy(ty_own.at[:, :, bsl], tyo_i, sem_idx),
            pltpu.make_async_copy(st_own.at[:, bsl], sto_i, sem_idx),
            pltpu.make_async_copy(mv_opp.at[:, :, bsl], mvp_i, sem_idx),
            pltpu.make_async_copy(it_opp.at[:, :, bsl], itp_i, sem_idx),
            pltpu.make_async_copy(ab_opp.at[:, :, bsl], abp_i, sem_idx),
            pltpu.make_async_copy(ty_opp.at[:, :, bsl], typ_i, sem_idx),
            pltpu.make_async_copy(st_opp.at[:, bsl], stp_i, sem_idx),
            pltpu.make_async_copy(fld.at[:, bsl], hd_i.at[pl.ds(0, 3)],
                                  sem_idx),
            pltpu.make_async_copy(sd.at[:, bsl], hd_i.at[pl.ds(3, 4)],
                                  sem_idx),
            pltpu.make_async_copy(osd.at[:, bsl], hd_i.at[pl.ds(7, 4)],
                                  sem_idx),
        ]
        for dsc in idx_descs:
            dsc.start()
        for dsc in idx_descs:
            dsc.wait()

        lookup(tab_fl, hd_i, [0, 1, 2], FIELD_DIM, 0, 1.0)
        lookup(tab_sd, hd_i, [3, 4, 5, 6], SIDE_DIM, 18, 1.0)
        lookup(tab_sd, hd_i, [7, 8, 9, 10], SIDE_DIM, 36, 1.0)
        pltpu.sync_copy(dyn.at[pl.ds(0, 2), bsl], tile.at[pl.ds(16, 2), pl.ds(0, BC)])
        pltpu.sync_copy(dyn.at[pl.ds(2, 2), bsl], tile.at[pl.ds(34, 2), pl.ds(0, BC)])
        hm_descs = [
            pltpu.make_async_copy(tile.at[pl.ds(0, 34), pl.ds(0, BC)],
                                  out.at[pl.ds(0, 34), bsl], sem_out),
            pltpu.make_async_copy(tile.at[pl.ds(34, 18), pl.ds(0, BC)],
                                  out.at[pl.ds(MID_BASE, 18), bsl], sem_out),
        ]
        for dsc in hm_descs:
            dsc.start()
        for dsc in hm_descs:
            dsc.wait()

        start_gathers(mvo_i, ito_i, 0, 0, sem_ga)

        def own_body(p, _):
            stage(p, bsl, True, jnp.bool_(True))
            return 0
        lax.fori_loop(0, 6, own_body, 0)
        out_desc(OWN_PER, OWN_BASE + 5 * OWN_PER, bsl).wait()

        def opp_body(p, _):
            stage(p, bsl, False, jnp.bool_(False))
            return 0
        lax.fori_loop(0, 6, opp_body, 0)
        out_desc(OPP_PER, OPP_BASE + 5 * OPP_PER, bsl).wait()
        return 0

    lax.fori_loop(0, NSUB, sub_body, 0)


@jax.jit
def kernel(own_move_idx, own_item_idx, own_ability_idx, own_type_idx,
           own_status_idx, opp_move_idx, opp_item_idx, opp_ability_idx,
           opp_type_idx, opp_status_idx, field_attrib_idx, side_attrib_idx,
           opp_side_attrib_idx, own_stats, own_boosts, opp_stats, opp_boosts,
           dyn_flags, W_move, W_item, W_ability, W_side, W_field, W_type,
           W_status):
    t3 = lambda a: jnp.transpose(a, (1, 2, 0)).astype(jnp.int32)
    f3 = lambda a: jnp.transpose(a, (1, 2, 0))
    mesh = plsc.VectorSubcoreMesh(core_axis_name="c", subcore_axis_name="s",
                                  num_cores=NC, num_subcores=NS)
    run = pl.kernel(
        _sc_encode,
        out_type=jax.ShapeDtypeStruct((D_OUT, B), jnp.float32),
        mesh=mesh,
        scratch_types=[
            pltpu.VMEM((1000, ABILITY_DIM + 1), jnp.float32),
            pltpu.VMEM((64, SIDE_DIM + 1), jnp.float32),
            pltpu.VMEM((64, FIELD_DIM + 1), jnp.float32),
            pltpu.VMEM((32, TYPE_DIM + 1), jnp.float32),
            pltpu.VMEM((16, STATUS_DIM + 1), jnp.float32),
            pltpu.VMEM((2, 6, BC, 32), jnp.float32),
            pltpu.VMEM((6, 4, BC), jnp.int32),
            pltpu.VMEM((6, 2, BC), jnp.int32),
            pltpu.VMEM((6, 2, BC), jnp.int32),
            pltpu.VMEM((6, 2, BC), jnp.int32),
            pltpu.VMEM((6, BC), jnp.int32),
            pltpu.VMEM((6, 4, BC), jnp.int32),
            pltpu.VMEM((6, 2, BC), jnp.int32),
            pltpu.VMEM((6, 2, BC), jnp.int32),
            pltpu.VMEM((6, 2, BC), jnp.int32),
            pltpu.VMEM((6, BC), jnp.int32),
            pltpu.VMEM((12, BC), jnp.int32),
            pltpu.VMEM((OWN_PER, BC + 1), jnp.float32),
            pltpu.SemaphoreType.DMA,
            pltpu.SemaphoreType.DMA,
            pltpu.SemaphoreType.DMA,
            pltpu.SemaphoreType.DMA,
            pltpu.SemaphoreType.DMA,
        ],
        compiler_params=pltpu.CompilerParams(use_tc_tiling_on_sc=False,
                                             needs_layout_passes=False),
    )
    return run(
        t3(own_move_idx), t3(own_item_idx), t3(own_ability_idx),
        t3(own_type_idx), jnp.transpose(own_status_idx).astype(jnp.int32),
        t3(opp_move_idx), t3(opp_item_idx), t3(opp_ability_idx),
        t3(opp_type_idx), jnp.transpose(opp_status_idx).astype(jnp.int32),
        jnp.transpose(field_attrib_idx).astype(jnp.int32),
        jnp.transpose(side_attrib_idx).astype(jnp.int32),
        jnp.transpose(opp_side_attrib_idx).astype(jnp.int32),
        f3(own_stats), f3(own_boosts), f3(opp_stats), f3(opp_boosts),
        jnp.transpose(dyn_flags),
        W_move, W_item, W_ability, W_side, W_field, W_type, W_status)
